# Initial kernel scaffold; baseline (speedup 1.0000x reference)
import jax
import jax.numpy as jnp
from jax.experimental import pallas as pl


def kernel(node_feats, batch_idx, W_attn, b_attn, W_mask, b_mask):
    raise NotImplementedError("write your pallas kernel here")


# speedup vs baseline: 2.0231x; 2.0231x over previous
import functools

import jax
import jax.numpy as jnp
from jax import lax
from jax.experimental import pallas as pl
from jax.experimental.pallas import tpu as pltpu
from jax.experimental.pallas import tpu_sc as plsc

N = 100000
D = 128
S = 64

NC = 2
NS = 16
NW = NC * NS

B = 400
NB = N // B
CH = 100
NCH = B // CH

_MESH = plsc.VectorSubcoreMesh(
    core_axis_name="c", subcore_axis_name="s", num_cores=NC, num_subcores=NS
)


@functools.partial(
    pl.kernel,
    out_type=jax.ShapeDtypeStruct((NC, S, D), jnp.float32),
    mesh=_MESH,
    compiler_params=pltpu.CompilerParams(needs_layout_passes=False),
    scratch_types=[
        pltpu.VMEM((B, D), jnp.float32),
        pltpu.VMEM((NCH, CH), jnp.int32),
        pltpu.VMEM((D,), jnp.float32),
        pltpu.VMEM((D,), jnp.float32),
        pltpu.VMEM((16,), jnp.float32),
        pltpu.VMEM((S, D), jnp.float32),
        pltpu.VMEM_SHARED((S, D), jnp.float32),
    ],
)
def _sc_pool(x_hbm, idx_hbm, wa_hbm, wm_hbm, b_hbm, out_hbm,
             xbuf, idxbuf, wav, wmv, bv, zbuf, acc):
    cid = lax.axis_index("c")
    sid = lax.axis_index("s")
    wid = sid * NC + cid

    pltpu.sync_copy(wa_hbm, wav)
    pltpu.sync_copy(wm_hbm, wmv)
    pltpu.sync_copy(b_hbm, bv)

    @pl.when(sid == 0)
    def _init():
        zeros16 = jnp.zeros((16,), jnp.float32)

        def zbody(i, carry):
            zbuf[i // (D // 16), pl.ds((i % (D // 16)) * 16, 16)] = zeros16
            return carry

        lax.fori_loop(0, S * (D // 16), zbody, 0)
        pltpu.sync_copy(zbuf, acc)

    plsc.subcore_barrier()

    wa_k = [wav[pl.ds(k * 16, 16)] for k in range(D // 16)]
    wm_k = [wmv[pl.ds(k * 16, 16)] for k in range(D // 16)]
    bvec = bv[...]
    ba = bvec[0]
    bm = bvec[1]

    def block_body(j, carry):
        b = wid + j * NW
        pltpu.sync_copy(x_hbm.at[pl.ds(b * B, B)], xbuf)
        pltpu.sync_copy(idx_hbm.at[pl.ds(b * NCH, NCH)], idxbuf)

        def row_body(r, rcarry):
            xk = [xbuf[r, pl.ds(k * 16, 16)] for k in range(D // 16)]
            pa = xk[0] * wa_k[0]
            pm = xk[0] * wm_k[0]
            for k in range(1, D // 16):
                pa = pa + xk[k] * wa_k[k]
                pm = pm + xk[k] * wm_k[k]
            pa_s = jnp.sum(pa) + ba
            pm_s = jnp.sum(pm) + bm
            sv = jnp.full((16,), pa_s, jnp.float32)
            mv = jnp.full((16,), pm_s, jnp.float32)
            w = sv / (1.0 + jnp.exp(-mv))
            for k in range(D // 16):
                xbuf[r, pl.ds(k * 16, 16)] = xk[k] * w
            return rcarry

        lax.fori_loop(0, B, row_body, 0)

        for c in range(NCH):
            pltpu.sync_copy(
                xbuf.at[pl.ds(c * CH, CH)], acc.at[idxbuf.at[c]], add=True
            )
        return carry

    nb_w = (NB - wid + NW - 1) // NW
    lax.fori_loop(0, nb_w, block_body, 0)

    plsc.subcore_barrier()

    @pl.when(sid == 0)
    def _writeout():
        pltpu.sync_copy(acc, out_hbm.at[cid])


def _combine_body(p_ref, o_ref):
    o_ref[...] = p_ref[0] + p_ref[1]


_combine = pl.pallas_call(
    _combine_body,
    out_shape=jax.ShapeDtypeStruct((S, D), jnp.float32),
)


@jax.jit
def kernel(node_feats, batch_idx, W_attn, b_attn, W_mask, b_mask):
    idx = batch_idx.astype(jnp.int32).reshape(N // CH, CH)
    wa = W_attn.reshape(D)
    wm = W_mask.reshape(D)
    bias = jnp.concatenate(
        [b_attn.astype(jnp.float32), b_mask.astype(jnp.float32),
         jnp.zeros((14,), jnp.float32)]
    )
    partials = _sc_pool(node_feats, idx, wa, wm, bias)
    return _combine(partials)


# speedup vs baseline: 2.4083x; 1.1904x over previous
import functools

import jax
import jax.numpy as jnp
from jax import lax
from jax.experimental import pallas as pl
from jax.experimental.pallas import tpu as pltpu
from jax.experimental.pallas import tpu_sc as plsc

N = 100000
D = 128
S = 64

NC = 2
NS = 16
NW = NC * NS

B = 400
NB = N // B
CH = 100
NCH = B // CH

_MESH = plsc.VectorSubcoreMesh(
    core_axis_name="c", subcore_axis_name="s", num_cores=NC, num_subcores=NS
)


@functools.partial(
    pl.kernel,
    out_type=jax.ShapeDtypeStruct((NC, S, D), jnp.float32),
    mesh=_MESH,
    compiler_params=pltpu.CompilerParams(needs_layout_passes=False),
    scratch_types=[
        pltpu.VMEM((B, D), jnp.float32),
        pltpu.VMEM((B, D), jnp.float32),
        pltpu.VMEM((NCH, CH), jnp.int32),
        pltpu.VMEM((D,), jnp.float32),
        pltpu.VMEM((D,), jnp.float32),
        pltpu.VMEM((16,), jnp.float32),
        pltpu.VMEM((S, D), jnp.float32),
        pltpu.VMEM_SHARED((S, D), jnp.float32),
    ],
)
def _sc_pool(x_hbm, idx_hbm, wa_hbm, wm_hbm, b_hbm, out_hbm,
             xbuf, wbuf, idxbuf, wav, wmv, bv, zbuf, acc):
    cid = lax.axis_index("c")
    sid = lax.axis_index("s")
    wid = sid * NC + cid

    pltpu.sync_copy(wa_hbm, wav)
    pltpu.sync_copy(wm_hbm, wmv)
    pltpu.sync_copy(b_hbm, bv)

    @pl.when(sid == 0)
    def _init():
        zeros16 = jnp.zeros((16,), jnp.float32)

        def zbody(i, carry):
            zbuf[i // (D // 16), pl.ds((i % (D // 16)) * 16, 16)] = zeros16
            return carry

        lax.fori_loop(0, S * (D // 16), zbody, 0)
        pltpu.sync_copy(zbuf, acc)

    plsc.subcore_barrier()

    wa_k = [wav[pl.ds(k * 16, 16)] for k in range(D // 16)]
    wm_k = [wmv[pl.ds(k * 16, 16)] for k in range(D // 16)]
    bvec = bv[...]
    ba = bvec[0]
    bm = bvec[1]

    def block_body(j, carry):
        b = wid + j * NW
        pltpu.sync_copy(x_hbm.at[pl.ds(b * B, B)], xbuf)
        pltpu.sync_copy(idx_hbm.at[pl.ds(b * NCH, NCH)], idxbuf)

        @plsc.parallel_loop(0, B, unroll=4)
        def _rows(r):
            xk = [xbuf[r, pl.ds(k * 16, 16)] for k in range(D // 16)]
            pa = xk[0] * wa_k[0]
            pm = xk[0] * wm_k[0]
            for k in range(1, D // 16):
                pa = pa + xk[k] * wa_k[k]
                pm = pm + xk[k] * wm_k[k]
            pa_s = jnp.sum(pa) + ba
            pm_s = jnp.sum(pm) + bm
            sv = jnp.full((16,), pa_s, jnp.float32)
            mv = jnp.full((16,), pm_s, jnp.float32)
            w = sv / (1.0 + jnp.exp(-mv))
            for k in range(D // 16):
                wbuf[r, pl.ds(k * 16, 16)] = xk[k] * w

        for c in range(NCH):
            pltpu.sync_copy(
                wbuf.at[pl.ds(c * CH, CH)], acc.at[idxbuf.at[c]], add=True
            )
        return carry

    nb_w = (NB - wid + NW - 1) // NW
    lax.fori_loop(0, nb_w, block_body, 0)

    plsc.subcore_barrier()

    @pl.when(sid == 0)
    def _writeout():
        pltpu.sync_copy(acc, out_hbm.at[cid])


def _combine_body(p_ref, o_ref):
    o_ref[...] = p_ref[0] + p_ref[1]


_combine = pl.pallas_call(
    _combine_body,
    out_shape=jax.ShapeDtypeStruct((S, D), jnp.float32),
)


@jax.jit
def kernel(node_feats, batch_idx, W_attn, b_attn, W_mask, b_mask):
    idx = batch_idx.astype(jnp.int32).reshape(N // CH, CH)
    wa = W_attn.reshape(D)
    wm = W_mask.reshape(D)
    bias = jnp.concatenate(
        [b_attn.astype(jnp.float32), b_mask.astype(jnp.float32),
         jnp.zeros((14,), jnp.float32)]
    )
    partials = _sc_pool(node_feats, idx, wa, wm, bias)
    return _combine(partials)


# speedup vs baseline: 3.0981x; 1.2864x over previous
import functools

import jax
import jax.numpy as jnp
from jax import lax
from jax.experimental import pallas as pl
from jax.experimental.pallas import tpu as pltpu
from jax.experimental.pallas import tpu_sc as plsc

N = 100000
D = 128
S = 64

NC = 2
NS = 16
NW = NC * NS

B = 160
NB = N // B
CH = 80
NCH = B // CH
NBJ = 5

_MESH = plsc.VectorSubcoreMesh(
    core_axis_name="c", subcore_axis_name="s", num_cores=NC, num_subcores=NS
)


@functools.partial(
    pl.kernel,
    out_type=jax.ShapeDtypeStruct((NC, S, D), jnp.float32),
    mesh=_MESH,
    compiler_params=pltpu.CompilerParams(needs_layout_passes=False),
    scratch_types=[
        [pltpu.VMEM((B, D), jnp.float32) for _ in range(2)],
        [pltpu.VMEM((B, D), jnp.float32) for _ in range(4)],
        [pltpu.VMEM((NCH, CH), jnp.int32) for _ in range(4)],
        pltpu.VMEM((D,), jnp.float32),
        pltpu.VMEM((D,), jnp.float32),
        pltpu.VMEM((16,), jnp.float32),
        [pltpu.SemaphoreType.DMA for _ in range(2)],
        [pltpu.SemaphoreType.DMA for _ in range(4)],
        pltpu.VMEM_SHARED((S, D), jnp.float32),
    ],
)
def _sc_pool(x_hbm, idx_hbm, wa_hbm, wm_hbm, b_hbm, out_hbm,
             xb, wb, ib, wav, wmv, bv, semx, sems, acc):
    cid = lax.axis_index("c")
    sid = lax.axis_index("s")
    wid = sid * NC + cid

    pltpu.sync_copy(wa_hbm, wav)
    pltpu.sync_copy(wm_hbm, wmv)
    pltpu.sync_copy(b_hbm, bv)

    @pl.when(sid == 0)
    def _init():
        zeros16 = jnp.zeros((16,), jnp.float32)

        def zbody(i, carry):
            wb[0][i // (D // 16), pl.ds((i % (D // 16)) * 16, 16)] = zeros16
            return carry

        lax.fori_loop(0, S * (D // 16), zbody, 0)
        pltpu.sync_copy(wb[0].at[pl.ds(0, S)], acc)

    plsc.subcore_barrier()

    wa_k = [wav[pl.ds(k * 16, 16)] for k in range(D // 16)]
    wm_k = [wmv[pl.ds(k * 16, 16)] for k in range(D // 16)]
    bvec = bv[...]
    ba = bvec[0]
    bm = bvec[1]

    nb_w = (NB - wid + NW - 1) // NW

    def _compute(xsrc, wdst):
        @plsc.parallel_loop(0, B, unroll=4)
        def _rows(r):
            xk = [xsrc[r, pl.ds(k * 16, 16)] for k in range(D // 16)]
            pa = xk[0] * wa_k[0]
            pm = xk[0] * wm_k[0]
            for k in range(1, D // 16):
                pa = pa + xk[k] * wa_k[k]
                pm = pm + xk[k] * wm_k[k]
            pa_s = jnp.sum(pa) + ba
            pm_s = jnp.sum(pm) + bm
            sv = jnp.full((16,), pa_s, jnp.float32)
            mv = jnp.full((16,), pm_s, jnp.float32)
            w = sv / (1.0 + jnp.exp(-mv))
            for k in range(D // 16):
                wdst[r, pl.ds(k * 16, 16)] = xk[k] * w

    def _start_in(blk, t2, t4):
        pltpu.async_copy(x_hbm.at[pl.ds(blk * B, B)], xb[t2], semx[t2])
        pltpu.async_copy(idx_hbm.at[pl.ds(blk * NCH, NCH)], ib[t4], semx[t2])

    def _wait_in(blk, t2, t4):
        pltpu.make_async_copy(
            x_hbm.at[pl.ds(blk * B, B)], xb[t2], semx[t2]).wait()
        pltpu.make_async_copy(
            idx_hbm.at[pl.ds(blk * NCH, NCH)], ib[t4], semx[t2]).wait()

    def _wait_scatter(t4):
        for c in range(NCH):
            pltpu.make_async_copy(
                wb[t4].at[pl.ds(c * CH, CH)], acc.at[ib[t4].at[c]],
                sems[t4]).wait()

    _start_in(wid, 0, 0)

    def outer(jj, carry):
        for t in range(4):
            j = jj * 4 + t
            b = wid + j * NW
            nxt = j + 1
            bn = wid + nxt * NW
            t2, t4 = t % 2, t
            n2, n4 = (t + 1) % 2, (t + 1) % 4

            @pl.when((j >= 3) & (nxt < nb_w))
            def _free_next():
                _wait_scatter(n4)

            @pl.when(nxt < nb_w)
            def _prefetch():
                _start_in(bn, n2, n4)

            @pl.when(j < nb_w)
            def _work():
                _wait_in(b, t2, t4)
                _compute(xb[t2], wb[t4])
                for c in range(NCH):
                    pltpu.async_copy(
                        wb[t4].at[pl.ds(c * CH, CH)], acc.at[ib[t4].at[c]],
                        sems[t4], add=True)
        return carry

    lax.fori_loop(0, NBJ, outer, 0)

    for p in range(4):
        _wait_scatter(p)

    plsc.subcore_barrier()

    @pl.when(sid == 0)
    def _writeout():
        pltpu.sync_copy(acc, out_hbm.at[cid])


def _combine_body(p_ref, o_ref):
    o_ref[...] = p_ref[0] + p_ref[1]


_combine = pl.pallas_call(
    _combine_body,
    out_shape=jax.ShapeDtypeStruct((S, D), jnp.float32),
)


@jax.jit
def kernel(node_feats, batch_idx, W_attn, b_attn, W_mask, b_mask):
    idx = batch_idx.astype(jnp.int32).reshape(N // CH, CH)
    wa = W_attn.reshape(D)
    wm = W_mask.reshape(D)
    bias = jnp.concatenate(
        [b_attn.astype(jnp.float32), b_mask.astype(jnp.float32),
         jnp.zeros((14,), jnp.float32)]
    )
    partials = _sc_pool(node_feats, idx, wa, wm, bias)
    return _combine(partials)


# speedup vs baseline: 5.1279x; 1.6552x over previous
import functools

import jax
import jax.numpy as jnp
from jax import lax
from jax.experimental import pallas as pl
from jax.experimental.pallas import tpu as pltpu
from jax.experimental.pallas import tpu_sc as plsc

N = 100000
D = 128
S = 64

NC = 2
NS = 16
NW = NC * NS

B = 80
NB = N // B
CH = 80
NCH = B // CH
NBJ = 10

_MESH = plsc.VectorSubcoreMesh(
    core_axis_name="c", subcore_axis_name="s", num_cores=NC, num_subcores=NS
)


@functools.partial(
    pl.kernel,
    out_type=jax.ShapeDtypeStruct((NC, S, D), jnp.float32),
    mesh=_MESH,
    compiler_params=pltpu.CompilerParams(needs_layout_passes=False),
    scratch_types=[
        [pltpu.VMEM((B, D), jnp.float32) for _ in range(2)],
        [pltpu.VMEM((B, D), jnp.float32) for _ in range(4)],
        [pltpu.VMEM((NCH, CH), jnp.int32) for _ in range(4)],
        pltpu.VMEM((D,), jnp.float32),
        pltpu.VMEM((D,), jnp.float32),
        pltpu.VMEM((16,), jnp.float32),
        pltpu.VMEM((B, 16), jnp.float32),
        [pltpu.SemaphoreType.DMA for _ in range(2)],
        [pltpu.SemaphoreType.DMA for _ in range(4)],
        pltpu.VMEM_SHARED((S, D), jnp.float32),
    ],
)
def _sc_pool(x_hbm, idx_hbm, wa_hbm, wm_hbm, b_hbm, out_hbm,
             xb, wb, ib, wav, wmv, bv, sbuf, semx, sems, acc):
    cid = lax.axis_index("c")
    sid = lax.axis_index("s")
    wid = sid * NC + cid

    pltpu.sync_copy(wa_hbm, wav)
    pltpu.sync_copy(wm_hbm, wmv)
    pltpu.sync_copy(b_hbm, bv)

    @pl.when(sid == 0)
    def _init():
        zeros16 = jnp.zeros((16,), jnp.float32)

        def zbody(i, carry):
            wb[0][i // (D // 16), pl.ds((i % (D // 16)) * 16, 16)] = zeros16
            return carry

        lax.fori_loop(0, S * (D // 16), zbody, 0)
        pltpu.sync_copy(wb[0].at[pl.ds(0, S)], acc)

    plsc.subcore_barrier()

    wa_k = [wav[pl.ds(k * 16, 16)] for k in range(D // 16)]
    wm_k = [wmv[pl.ds(k * 16, 16)] for k in range(D // 16)]
    bvec = bv[...]
    ba = bvec[0]
    bm = bvec[1]

    nb_w = (NB - wid + NW - 1) // NW

    def _compute(xsrc, wdst):
        @plsc.parallel_loop(0, B, unroll=8)
        def _score(r):
            x0 = xsrc[r, pl.ds(0, 16)]
            pa = x0 * wa_k[0]
            pm = x0 * wm_k[0]
            for k in range(1, D // 16):
                xk = xsrc[r, pl.ds(k * 16, 16)]
                pa = pa + xk * wa_k[k]
                pm = pm + xk * wm_k[k]
            pa_s = jnp.sum(pa) + ba
            pm_s = jnp.sum(pm) + bm
            sv = jnp.full((16,), pa_s, jnp.float32)
            mv = jnp.full((16,), pm_s, jnp.float32)
            sbuf[r, :] = sv / (1.0 + jnp.exp(-mv))

        @plsc.parallel_loop(0, B, unroll=4)
        def _scale(r):
            w = sbuf[r, :]
            for k in range(D // 16):
                wdst[r, pl.ds(k * 16, 16)] = xsrc[r, pl.ds(k * 16, 16)] * w

    def _start_in(blk, t2, t4):
        pltpu.async_copy(x_hbm.at[pl.ds(blk * B, B)], xb[t2], semx[t2])
        pltpu.async_copy(idx_hbm.at[pl.ds(blk * NCH, NCH)], ib[t4], semx[t2])

    def _wait_in(blk, t2, t4):
        pltpu.make_async_copy(
            x_hbm.at[pl.ds(blk * B, B)], xb[t2], semx[t2]).wait()
        pltpu.make_async_copy(
            idx_hbm.at[pl.ds(blk * NCH, NCH)], ib[t4], semx[t2]).wait()

    def _wait_scatter(t4):
        for c in range(NCH):
            pltpu.make_async_copy(
                wb[t4].at[pl.ds(c * CH, CH)], acc.at[ib[t4].at[c]],
                sems[t4]).wait()

    _start_in(wid, 0, 0)

    def outer(jj, carry):
        for t in range(4):
            j = jj * 4 + t
            b = wid + j * NW
            nxt = j + 1
            bn = wid + nxt * NW
            t2, t4 = t % 2, t
            n2, n4 = (t + 1) % 2, (t + 1) % 4

            @pl.when((j >= 3) & (nxt < nb_w))
            def _free_next():
                _wait_scatter(n4)

            @pl.when(nxt < nb_w)
            def _prefetch():
                _start_in(bn, n2, n4)

            @pl.when(j < nb_w)
            def _work():
                _wait_in(b, t2, t4)
                _compute(xb[t2], wb[t4])
                for c in range(NCH):
                    pltpu.async_copy(
                        wb[t4].at[pl.ds(c * CH, CH)], acc.at[ib[t4].at[c]],
                        sems[t4], add=True)
        return carry

    lax.fori_loop(0, NBJ, outer, 0)

    for p in range(4):
        _wait_scatter(p)

    plsc.subcore_barrier()

    @pl.when(sid == 0)
    def _writeout():
        pltpu.sync_copy(acc, out_hbm.at[cid])


def _combine_body(p_ref, o_ref):
    o_ref[...] = p_ref[0] + p_ref[1]


_combine = pl.pallas_call(
    _combine_body,
    out_shape=jax.ShapeDtypeStruct((S, D), jnp.float32),
)


@jax.jit
def kernel(node_feats, batch_idx, W_attn, b_attn, W_mask, b_mask):
    idx = batch_idx.astype(jnp.int32).reshape(N // CH, CH)
    wa = W_attn.reshape(D)
    wm = W_mask.reshape(D)
    bias = jnp.concatenate(
        [b_attn.astype(jnp.float32), b_mask.astype(jnp.float32),
         jnp.zeros((14,), jnp.float32)]
    )
    partials = _sc_pool(node_feats, idx, wa, wm, bias)
    return _combine(partials)


# speedup vs baseline: 5.5928x; 1.0907x over previous
import functools

import jax
import jax.numpy as jnp
from jax import lax
from jax.experimental import pallas as pl
from jax.experimental.pallas import tpu as pltpu
from jax.experimental.pallas import tpu_sc as plsc

N = 100000
D = 128
S = 64

NC = 2
NS = 16
NW = NC * NS

B = 160
NB = N // B
CH = 80
NCH = B // CH
NBJ = 5

_MESH = plsc.VectorSubcoreMesh(
    core_axis_name="c", subcore_axis_name="s", num_cores=NC, num_subcores=NS
)


@functools.partial(
    pl.kernel,
    out_type=jax.ShapeDtypeStruct((NC, S, D), jnp.float32),
    mesh=_MESH,
    compiler_params=pltpu.CompilerParams(needs_layout_passes=False),
    scratch_types=[
        [pltpu.VMEM((B, D), jnp.float32) for _ in range(4)],
        [pltpu.VMEM((NCH, CH), jnp.int32) for _ in range(4)],
        pltpu.VMEM((D,), jnp.float32),
        pltpu.VMEM((D,), jnp.float32),
        pltpu.VMEM((16,), jnp.float32),
        pltpu.VMEM((B, 16), jnp.float32),
        pltpu.VMEM((S, D), jnp.float32),
        [pltpu.SemaphoreType.DMA for _ in range(4)],
        [pltpu.SemaphoreType.DMA for _ in range(4)],
        pltpu.VMEM_SHARED((S, D), jnp.float32),
    ],
)
def _sc_pool(x_hbm, idx_hbm, wa_hbm, wm_hbm, b_hbm, out_hbm,
             xb, ib, wav, wmv, bv, sbuf, zbuf, semx, sems, acc):
    cid = lax.axis_index("c")
    sid = lax.axis_index("s")
    wid = sid * NC + cid

    pltpu.sync_copy(wa_hbm, wav)
    pltpu.sync_copy(wm_hbm, wmv)
    pltpu.sync_copy(b_hbm, bv)

    @pl.when(sid == 0)
    def _init():
        zeros16 = jnp.zeros((16,), jnp.float32)

        def zbody(i, carry):
            zbuf[i // (D // 16), pl.ds((i % (D // 16)) * 16, 16)] = zeros16
            return carry

        lax.fori_loop(0, S * (D // 16), zbody, 0)
        pltpu.sync_copy(zbuf, acc)

    plsc.subcore_barrier()

    wa_k = [wav[pl.ds(k * 16, 16)] for k in range(D // 16)]
    wm_k = [wmv[pl.ds(k * 16, 16)] for k in range(D // 16)]
    bvec = bv[...]
    ba = bvec[0]
    bm = bvec[1]

    nb_w = (NB - wid + NW - 1) // NW

    def _compute(xsrc):
        @plsc.parallel_loop(0, B, unroll=8)
        def _score(r):
            x0 = xsrc[r, pl.ds(0, 16)]
            pa = x0 * wa_k[0]
            pm = x0 * wm_k[0]
            for k in range(1, D // 16):
                xk = xsrc[r, pl.ds(k * 16, 16)]
                pa = pa + xk * wa_k[k]
                pm = pm + xk * wm_k[k]
            pa_s = jnp.sum(pa) + ba
            pm_s = jnp.sum(pm) + bm
            sv = jnp.full((16,), pa_s, jnp.float32)
            mv = jnp.full((16,), pm_s, jnp.float32)
            sbuf[r, :] = sv / (1.0 + jnp.exp(-mv))

        @plsc.parallel_loop(0, B, unroll=8)
        def _scale(r):
            w = sbuf[r, :]
            for k in range(D // 16):
                xsrc[r, pl.ds(k * 16, 16)] = xsrc[r, pl.ds(k * 16, 16)] * w

    def _start_in(blk, t4):
        pltpu.async_copy(x_hbm.at[pl.ds(blk * B, B)], xb[t4], semx[t4])
        pltpu.async_copy(idx_hbm.at[pl.ds(blk * NCH, NCH)], ib[t4], semx[t4])

    def _wait_in(blk, t4):
        pltpu.make_async_copy(
            x_hbm.at[pl.ds(blk * B, B)], xb[t4], semx[t4]).wait()
        pltpu.make_async_copy(
            idx_hbm.at[pl.ds(blk * NCH, NCH)], ib[t4], semx[t4]).wait()

    def _wait_scatter(t4):
        for c in range(NCH):
            pltpu.make_async_copy(
                xb[t4].at[pl.ds(c * CH, CH)], acc.at[ib[t4].at[c]],
                sems[t4]).wait()

    _start_in(wid, 0)

    def outer(jj, carry):
        for t in range(4):
            j = jj * 4 + t
            b = wid + j * NW
            nxt = j + 1
            bn = wid + nxt * NW
            n4 = (t + 1) % 4

            @pl.when((j >= 3) & (nxt < nb_w))
            def _free_next():
                _wait_scatter(n4)

            @pl.when(nxt < nb_w)
            def _prefetch():
                _start_in(bn, n4)

            @pl.when(j < nb_w)
            def _work():
                _wait_in(b, t)
                _compute(xb[t])
                for c in range(NCH):
                    pltpu.async_copy(
                        xb[t].at[pl.ds(c * CH, CH)], acc.at[ib[t].at[c]],
                        sems[t], add=True)
        return carry

    lax.fori_loop(0, NBJ, outer, 0)

    for p in range(4):
        _wait_scatter(p)

    plsc.subcore_barrier()

    @pl.when(sid == 0)
    def _writeout():
        pltpu.sync_copy(acc, out_hbm.at[cid])


def _combine_body(p_ref, o_ref):
    o_ref[...] = p_ref[0] + p_ref[1]


_combine = pl.pallas_call(
    _combine_body,
    out_shape=jax.ShapeDtypeStruct((S, D), jnp.float32),
)


@jax.jit
def kernel(node_feats, batch_idx, W_attn, b_attn, W_mask, b_mask):
    idx = batch_idx.astype(jnp.int32).reshape(N // CH, CH)
    wa = W_attn.reshape(D)
    wm = W_mask.reshape(D)
    bias = jnp.concatenate(
        [b_attn.astype(jnp.float32), b_mask.astype(jnp.float32),
         jnp.zeros((14,), jnp.float32)]
    )
    partials = _sc_pool(node_feats, idx, wa, wm, bias)
    return _combine(partials)
